# Initial kernel scaffold; baseline (speedup 1.0000x reference)
import jax
import jax.numpy as jnp
from jax.experimental import pallas as pl


def kernel(x, params, modality_id):
    raise NotImplementedError("write your pallas kernel here")


# speedup vs baseline: 1.3501x; 1.3501x over previous
import jax
import jax.numpy as jnp
from jax.experimental import pallas as pl

TOK = 2048
D = 1024
GH = 512
E = 8
K = 4
EPS_BN = 1e-5
EPS_LN = 1e-5
TM = 256
NT = TOK // TM

def _dot(a, b):
    return jax.lax.dot_general(a.astype(jnp.bfloat16), b.astype(jnp.bfloat16),
                               (((1,), (0,)), ((), ())),
                               preferred_element_type=jnp.float32)


def _pass1a(x_ref, v_ref,
            w1t0_ref, b10_ref, w2t0_ref, b20_ref,
            w1t1_ref, b11_ref, w2t1_ref, b21_ref,
            w1t2_ref, b12_ref, w1t3_ref, b13_ref,
            w1ts_ref, b1s_ref, gs_ref, bes_ref, w2ts_ref, b2s_ref,
            shared_ref, priv_ref, w_ref, h2_ref, h3_ref, bns_ref, us_ref, aux_ref):
    i = pl.program_id(0)
    x = x_ref[...]
    v = v_ref[...]

    vmax = jnp.max(v, axis=1, keepdims=True)
    ev = jnp.exp(v - vmax)
    p_full = ev / jnp.sum(ev, axis=1, keepdims=True)

    @pl.when(i == 0)
    def _():
        us_ref[...] = jnp.zeros_like(us_ref)
        bns_ref[...] = jnp.zeros_like(bns_ref)

    us_ref[...] += jnp.sum(p_full, axis=0, keepdims=True)

    eidx = jax.lax.broadcasted_iota(jnp.int32, (1, E), 1)
    ranks = jnp.zeros_like(v)
    for ep in range(E):
        col = v[:, ep:ep + 1]
        ranks += ((col > v) | ((col == v) & (eidx > ep))).astype(v.dtype)
    sel = ranks < K
    ew = jnp.where(sel, ev, 0.0)
    w = ew / jnp.sum(ew, axis=1, keepdims=True)
    w_ref[...] = w

    h = jnp.maximum(_dot(x, w1t0_ref[...]) + b10_ref[...], 0.0)
    priv = w[:, 0:1] * (_dot(h, w2t0_ref[...]) + b20_ref[...])
    h = jnp.maximum(_dot(x, w1t1_ref[...]) + b11_ref[...], 0.0)
    priv += w[:, 1:2] * (_dot(h, w2t1_ref[...]) + b21_ref[...])

    h2 = jnp.tanh(_dot(x, w1t2_ref[...]) + b12_ref[...])
    h2_ref[...] = h2
    bns_ref[0:1, :] += jnp.sum(h2, axis=0, keepdims=True)
    bns_ref[1:2, :] += jnp.sum(h2 * h2, axis=0, keepdims=True)
    h3 = jnp.tanh(_dot(x, w1t3_ref[...]) + b13_ref[...])
    h3_ref[...] = h3
    bns_ref[2:3, :] += jnp.sum(h3, axis=0, keepdims=True)
    bns_ref[3:4, :] += jnp.sum(h3 * h3, axis=0, keepdims=True)

    priv_ref[...] = priv

    h = jnp.maximum(_dot(x, w1ts_ref[...]) + b1s_ref[...], 0.0)
    mu = jnp.mean(h, axis=1, keepdims=True)
    var = jnp.mean(h * h, axis=1, keepdims=True) - mu * mu
    hn = (h - mu) * jax.lax.rsqrt(var + EPS_LN) * gs_ref[...] + bes_ref[...]
    shared_ref[...] = _dot(hn, w2ts_ref[...]) + b2s_ref[...]

    @pl.when(i == NT - 1)
    def _():
        ideal = 1.0 / E
        usage = us_ref[...] * (1.0 / TOK)
        kl = jnp.sum(ideal * (jnp.log(ideal) - jnp.log(usage + 1e-10)))
        aux_ref[...] = (kl / E).reshape(1, 1)


def _pass1b(x_ref, w_ref, privp_ref,
            weff4_ref, beff4_ref, weff5_ref, beff5_ref,
            w1t6_ref, b16_ref, g6_ref, be6_ref, w2t6_ref, b26_ref,
            w1t7_ref, b17_ref, g7_ref, be7_ref, w2t7_ref, b27_ref,
            priv_ref):
    x = x_ref[...]
    w = w_ref[...]
    priv = privp_ref[...]

    z = _dot(x, weff4_ref[...]) + beff4_ref[...]
    priv += w[:, 4:5] * (0.5 * z * (1.0 + jax.lax.erf(z * 0.7071067811865476)))
    z = _dot(x, weff5_ref[...]) + beff5_ref[...]
    priv += w[:, 5:6] * (0.5 * z * (1.0 + jax.lax.erf(z * 0.7071067811865476)))

    for w1t_ref, b1_ref, gg_ref, be_ref, w2t_ref, b2_ref, c in (
            (w1t6_ref, b16_ref, g6_ref, be6_ref, w2t6_ref, b26_ref, 6),
            (w1t7_ref, b17_ref, g7_ref, be7_ref, w2t7_ref, b27_ref, 7)):
        z = _dot(x, w1t_ref[...]) + b1_ref[...]
        h = z * jax.nn.sigmoid(z)
        mu = jnp.mean(h, axis=1, keepdims=True)
        var = jnp.mean(h * h, axis=1, keepdims=True) - mu * mu
        hn = (h - mu) * jax.lax.rsqrt(var + EPS_LN) * gg_ref[...] + be_ref[...]
        priv += w[:, c:c + 1] * (_dot(hn, w2t_ref[...]) + b2_ref[...])

    priv_ref[...] = priv


def _pass2(h2_ref, h3_ref, bns_ref, w_ref, privp_ref,
           g2_ref, be2_ref, w2t2_ref, b22_ref,
           g3_ref, be3_ref, w2t3_ref, b23_ref,
           priv_ref):
    w = w_ref[...]
    priv = privp_ref[...]
    n = 1.0 / TOK
    for h_ref, r0, gg_ref, be_ref, w2t_ref, b2_ref, c in (
            (h2_ref, 0, g2_ref, be2_ref, w2t2_ref, b22_ref, 2),
            (h3_ref, 2, g3_ref, be3_ref, w2t3_ref, b23_ref, 3)):
        s = bns_ref[r0:r0 + 1, :]
        sq = bns_ref[r0 + 1:r0 + 2, :]
        mu = s * n
        var = sq * n - mu * mu
        hn = (h_ref[...] - mu) * jax.lax.rsqrt(var + EPS_BN) * gg_ref[...] + be_ref[...]
        priv += w[:, c:c + 1] * (_dot(hn, w2t_ref[...]) + b2_ref[...])
    priv_ref[...] = priv


def _row(b):
    return b.reshape(1, -1)


def _fold_conv(p):
    W = p["W"].reshape(D, 4, D)
    cw = p["cw"]
    t0 = jnp.pad(W[:, :, 1:], ((0, 0), (0, 0), (0, 1)))
    t2 = jnp.pad(W[:, :, :-1], ((0, 0), (0, 0), (1, 0)))
    weff = (cw[:, 0, 0][None, :, None] * t0
            + cw[:, 0, 1][None, :, None] * W
            + cw[:, 0, 2][None, :, None] * t2).sum(axis=1)
    beff = p["b"] + (p["cb"][None, :] * W.sum(axis=2)).sum(axis=1)
    return weff.T, _row(beff)


def kernel(x, params, modality_id):
    ex = params["experts"]
    gp = params["gate"]
    sp = params["shared"]
    mod = modality_id.astype(jnp.float32).reshape(TOK, 1)

    gi = jnp.concatenate([x, mod], axis=-1)
    v = (_dot(jnp.tanh(_dot(gi, gp["W1"].T) + gp["b1"]), gp["W2"].T)
         + gp["b2"]) * (1.0 / 0.7)
    v = jnp.clip(v, -10.0, 10.0)

    weff4, beff4 = _fold_conv(ex[4])
    weff5, beff5 = _fold_conv(ex[5])

    f32 = jnp.float32
    tok_d = pl.BlockSpec((TM, D), lambda i: (i, 0))
    tok_1 = pl.BlockSpec((TM, 1), lambda i: (i, 0))
    tok_e = pl.BlockSpec((TM, E), lambda i: (i, 0))

    def full(a):
        return pl.BlockSpec(a.shape, lambda i: tuple(0 for _ in a.shape))

    p1a_in = [
        x, v,
        ex[0]["W1"].T, _row(ex[0]["b1"]), ex[0]["W2"].T, _row(ex[0]["b2"]),
        ex[1]["W1"].T, _row(ex[1]["b1"]), ex[1]["W2"].T, _row(ex[1]["b2"]),
        ex[2]["W1"].T, _row(ex[2]["b1"]), ex[3]["W1"].T, _row(ex[3]["b1"]),
        sp["W1"].T, _row(sp["b1"]), _row(sp["g"]), _row(sp["be"]),
        sp["W2"].T, _row(sp["b2"]),
    ]
    p1a_specs = [tok_d, tok_e] + [full(a) for a in p1a_in[2:]]

    shared, priva, w, h2, h3, bns, us, aux = pl.pallas_call(
        _pass1a,
        grid=(NT,),
        in_specs=p1a_specs,
        out_specs=[tok_d, tok_d, tok_e, tok_d, tok_d,
                   pl.BlockSpec((8, D), lambda i: (0, 0)),
                   pl.BlockSpec((1, E), lambda i: (0, 0)),
                   pl.BlockSpec((1, 1), lambda i: (0, 0))],
        out_shape=[
            jax.ShapeDtypeStruct((TOK, D), f32),
            jax.ShapeDtypeStruct((TOK, D), f32),
            jax.ShapeDtypeStruct((TOK, E), f32),
            jax.ShapeDtypeStruct((TOK, D), f32),
            jax.ShapeDtypeStruct((TOK, D), f32),
            jax.ShapeDtypeStruct((8, D), f32),
            jax.ShapeDtypeStruct((1, E), f32),
            jax.ShapeDtypeStruct((1, 1), f32),
        ],
    )(*p1a_in)

    p1b_in = [
        x, w, priva,
        weff4, beff4, weff5, beff5,
        ex[6]["W1"].T, _row(ex[6]["b1"]), _row(ex[6]["g"]), _row(ex[6]["be"]),
        ex[6]["W2"].T, _row(ex[6]["b2"]),
        ex[7]["W1"].T, _row(ex[7]["b1"]), _row(ex[7]["g"]), _row(ex[7]["be"]),
        ex[7]["W2"].T, _row(ex[7]["b2"]),
    ]
    p1b_specs = [tok_d, tok_e, tok_d] + [full(a) for a in p1b_in[3:]]

    privp = pl.pallas_call(
        _pass1b,
        grid=(NT,),
        in_specs=p1b_specs,
        out_specs=tok_d,
        out_shape=jax.ShapeDtypeStruct((TOK, D), f32),
    )(*p1b_in)

    p2_in = [
        h2, h3, bns, w, privp,
        _row(ex[2]["g"]), _row(ex[2]["be"]), ex[2]["W2"].T, _row(ex[2]["b2"]),
        _row(ex[3]["g"]), _row(ex[3]["be"]), ex[3]["W2"].T, _row(ex[3]["b2"]),
    ]
    p2_specs = ([tok_d, tok_d, pl.BlockSpec((8, D), lambda i: (0, 0)), tok_e, tok_d]
                + [full(a) for a in p2_in[5:]])

    private = pl.pallas_call(
        _pass2,
        grid=(NT,),
        in_specs=p2_specs,
        out_specs=tok_d,
        out_shape=jax.ShapeDtypeStruct((TOK, D), f32),
    )(*p2_in)

    return (shared, private, aux.reshape(()))


# speedup vs baseline: 1.4841x; 1.0992x over previous
import jax
import jax.numpy as jnp
from jax.experimental import pallas as pl

TOK = 2048
D = 1024
E = 8
K = 4
EPS_BN = 1e-5
EPS_LN = 1e-5
TM = 256
NT = TOK // TM


def _dot(a, b):
    return jax.lax.dot_general(a.astype(jnp.bfloat16), b.astype(jnp.bfloat16),
                               (((1,), (0,)), ((), ())),
                               preferred_element_type=jnp.float32)


def _pass1(x_ref, v_ref,
           w1t0_ref, b10_ref, w2t0_ref, b20_ref,
           w1t1_ref, b11_ref, w2t1_ref, b21_ref,
           w1t2_ref, b12_ref, w1t3_ref, b13_ref,
           weff4_ref, beff4_ref, weff5_ref, beff5_ref,
           w1t6_ref, b16_ref, g6_ref, be6_ref, w2t6_ref, b26_ref,
           w1t7_ref, b17_ref, g7_ref, be7_ref, w2t7_ref, b27_ref,
           w1ts_ref, b1s_ref, gs_ref, bes_ref, w2ts_ref, b2s_ref,
           shared_ref, priv_ref, w_ref, h2_ref, h3_ref, bns_ref, us_ref, aux_ref):
    i = pl.program_id(0)
    x = x_ref[...]
    v = v_ref[...]

    vmax = jnp.max(v, axis=1, keepdims=True)
    ev = jnp.exp(v - vmax)
    p_full = ev / jnp.sum(ev, axis=1, keepdims=True)

    @pl.when(i == 0)
    def _():
        us_ref[...] = jnp.zeros_like(us_ref)
        bns_ref[...] = jnp.zeros_like(bns_ref)

    us_ref[...] += jnp.sum(p_full, axis=0, keepdims=True)

    eidx = jax.lax.broadcasted_iota(jnp.int32, (1, E), 1)
    ranks = jnp.zeros_like(v)
    for ep in range(E):
        col = v[:, ep:ep + 1]
        ranks += ((col > v) | ((col == v) & (eidx > ep))).astype(v.dtype)
    sel = ranks < K
    ew = jnp.where(sel, ev, 0.0)
    w = ew / jnp.sum(ew, axis=1, keepdims=True)
    w_ref[...] = w

    h = jnp.maximum(_dot(x, w1t0_ref[...]) + b10_ref[...], 0.0)
    priv = w[:, 0:1] * (_dot(h, w2t0_ref[...]) + b20_ref[...])
    h = jnp.maximum(_dot(x, w1t1_ref[...]) + b11_ref[...], 0.0)
    priv += w[:, 1:2] * (_dot(h, w2t1_ref[...]) + b21_ref[...])

    h2 = jnp.tanh(_dot(x, w1t2_ref[...]) + b12_ref[...])
    h2_ref[...] = h2.astype(jnp.bfloat16)
    bns_ref[0:1, :] += jnp.sum(h2, axis=0, keepdims=True)
    bns_ref[1:2, :] += jnp.sum(h2 * h2, axis=0, keepdims=True)
    h3 = jnp.tanh(_dot(x, w1t3_ref[...]) + b13_ref[...])
    h3_ref[...] = h3.astype(jnp.bfloat16)
    bns_ref[2:3, :] += jnp.sum(h3, axis=0, keepdims=True)
    bns_ref[3:4, :] += jnp.sum(h3 * h3, axis=0, keepdims=True)

    z = _dot(x, weff4_ref[...]) + beff4_ref[...]
    priv += w[:, 4:5] * (0.5 * z * (1.0 + jax.lax.erf(z * 0.7071067811865476)))
    z = _dot(x, weff5_ref[...]) + beff5_ref[...]
    priv += w[:, 5:6] * (0.5 * z * (1.0 + jax.lax.erf(z * 0.7071067811865476)))

    for w1t_ref, b1_ref, gg_ref, be_ref, w2t_ref, b2_ref, c in (
            (w1t6_ref, b16_ref, g6_ref, be6_ref, w2t6_ref, b26_ref, 6),
            (w1t7_ref, b17_ref, g7_ref, be7_ref, w2t7_ref, b27_ref, 7)):
        z = _dot(x, w1t_ref[...]) + b1_ref[...]
        h = z * jax.nn.sigmoid(z)
        mu = jnp.mean(h, axis=1, keepdims=True)
        var = jnp.mean(h * h, axis=1, keepdims=True) - mu * mu
        hn = (h - mu) * jax.lax.rsqrt(var + EPS_LN) * gg_ref[...] + be_ref[...]
        priv += w[:, c:c + 1] * (_dot(hn, w2t_ref[...]) + b2_ref[...])

    priv_ref[...] = priv

    h = jnp.maximum(_dot(x, w1ts_ref[...]) + b1s_ref[...], 0.0)
    mu = jnp.mean(h, axis=1, keepdims=True)
    var = jnp.mean(h * h, axis=1, keepdims=True) - mu * mu
    hn = (h - mu) * jax.lax.rsqrt(var + EPS_LN) * gs_ref[...] + bes_ref[...]
    shared_ref[...] = _dot(hn, w2ts_ref[...]) + b2s_ref[...]

    @pl.when(i == NT - 1)
    def _():
        ideal = 1.0 / E
        usage = us_ref[...] * (1.0 / TOK)
        kl = jnp.sum(ideal * (jnp.log(ideal) - jnp.log(usage + 1e-10)))
        aux_ref[...] = (kl / E).reshape(1, 1)


def _pass2(h2_ref, h3_ref, bns_ref, w_ref, privp_ref,
           g2_ref, be2_ref, w2t2_ref, b22_ref,
           g3_ref, be3_ref, w2t3_ref, b23_ref,
           priv_ref):
    w = w_ref[...]
    priv = privp_ref[...]
    n = 1.0 / TOK
    for h_ref, r0, gg_ref, be_ref, w2t_ref, b2_ref, c in (
            (h2_ref, 0, g2_ref, be2_ref, w2t2_ref, b22_ref, 2),
            (h3_ref, 2, g3_ref, be3_ref, w2t3_ref, b23_ref, 3)):
        s = bns_ref[r0:r0 + 1, :]
        sq = bns_ref[r0 + 1:r0 + 2, :]
        mu = s * n
        var = sq * n - mu * mu
        hn = ((h_ref[...].astype(jnp.float32) - mu)
              * jax.lax.rsqrt(var + EPS_BN) * gg_ref[...] + be_ref[...])
        priv += w[:, c:c + 1] * (_dot(hn, w2t_ref[...]) + b2_ref[...])
    priv_ref[...] = priv


def _row(b):
    return b.reshape(1, -1)


def _bf(a):
    return a.astype(jnp.bfloat16)


def _fold_conv(p):
    W = p["W"].reshape(D, 4, D)
    cw = p["cw"]
    t0 = jnp.pad(W[:, :, 1:], ((0, 0), (0, 0), (0, 1)))
    t2 = jnp.pad(W[:, :, :-1], ((0, 0), (0, 0), (1, 0)))
    weff = (cw[:, 0, 0][None, :, None] * t0
            + cw[:, 0, 1][None, :, None] * W
            + cw[:, 0, 2][None, :, None] * t2).sum(axis=1)
    beff = p["b"] + (p["cb"][None, :] * W.sum(axis=2)).sum(axis=1)
    return weff.T, _row(beff)


def kernel(x, params, modality_id):
    ex = params["experts"]
    gp = params["gate"]
    sp = params["shared"]
    mod = modality_id.astype(jnp.float32).reshape(TOK, 1)

    gi = jnp.concatenate([x, mod], axis=-1)
    v = (_dot(jnp.tanh(_dot(gi, gp["W1"].T) + gp["b1"]), gp["W2"].T)
         + gp["b2"]) * (1.0 / 0.7)
    v = jnp.clip(v, -10.0, 10.0)

    weff4, beff4 = _fold_conv(ex[4])
    weff5, beff5 = _fold_conv(ex[5])

    f32 = jnp.float32
    bf16 = jnp.bfloat16
    xb = _bf(x)
    tok_d = pl.BlockSpec((TM, D), lambda i: (i, 0))
    tok_e = pl.BlockSpec((TM, E), lambda i: (i, 0))

    def full(a):
        return pl.BlockSpec(a.shape, lambda i: tuple(0 for _ in a.shape))

    p1_in = [
        xb, v,
        _bf(ex[0]["W1"].T), _row(ex[0]["b1"]), _bf(ex[0]["W2"].T), _row(ex[0]["b2"]),
        _bf(ex[1]["W1"].T), _row(ex[1]["b1"]), _bf(ex[1]["W2"].T), _row(ex[1]["b2"]),
        _bf(ex[2]["W1"].T), _row(ex[2]["b1"]), _bf(ex[3]["W1"].T), _row(ex[3]["b1"]),
        _bf(weff4), beff4, _bf(weff5), beff5,
        _bf(ex[6]["W1"].T), _row(ex[6]["b1"]), _row(ex[6]["g"]), _row(ex[6]["be"]),
        _bf(ex[6]["W2"].T), _row(ex[6]["b2"]),
        _bf(ex[7]["W1"].T), _row(ex[7]["b1"]), _row(ex[7]["g"]), _row(ex[7]["be"]),
        _bf(ex[7]["W2"].T), _row(ex[7]["b2"]),
        _bf(sp["W1"].T), _row(sp["b1"]), _row(sp["g"]), _row(sp["be"]),
        _bf(sp["W2"].T), _row(sp["b2"]),
    ]
    p1_specs = [tok_d, tok_e] + [full(a) for a in p1_in[2:]]

    shared, privp, w, h2, h3, bns, us, aux = pl.pallas_call(
        _pass1,
        grid=(NT,),
        in_specs=p1_specs,
        out_specs=[tok_d, tok_d, tok_e, tok_d, tok_d,
                   pl.BlockSpec((8, D), lambda i: (0, 0)),
                   pl.BlockSpec((1, E), lambda i: (0, 0)),
                   pl.BlockSpec((1, 1), lambda i: (0, 0))],
        out_shape=[
            jax.ShapeDtypeStruct((TOK, D), f32),
            jax.ShapeDtypeStruct((TOK, D), f32),
            jax.ShapeDtypeStruct((TOK, E), f32),
            jax.ShapeDtypeStruct((TOK, D), bf16),
            jax.ShapeDtypeStruct((TOK, D), bf16),
            jax.ShapeDtypeStruct((8, D), f32),
            jax.ShapeDtypeStruct((1, E), f32),
            jax.ShapeDtypeStruct((1, 1), f32),
        ],
    )(*p1_in)

    p2_in = [
        h2, h3, bns, w, privp,
        _row(ex[2]["g"]), _row(ex[2]["be"]), _bf(ex[2]["W2"].T), _row(ex[2]["b2"]),
        _row(ex[3]["g"]), _row(ex[3]["be"]), _bf(ex[3]["W2"].T), _row(ex[3]["b2"]),
    ]
    p2_specs = ([tok_d, tok_d, pl.BlockSpec((8, D), lambda i: (0, 0)), tok_e, tok_d]
                + [full(a) for a in p2_in[5:]])

    private = pl.pallas_call(
        _pass2,
        grid=(NT,),
        in_specs=p2_specs,
        out_specs=tok_d,
        out_shape=jax.ShapeDtypeStruct((TOK, D), f32),
    )(*p2_in)

    return (shared, private, aux.reshape(()))
